# baseline (speedup 1.0000x reference)
import functools

import jax
import jax.numpy as jnp
from jax.experimental import pallas as pl
from jax.experimental.pallas import tpu as pltpu

_TOPK = 64
_TOPK_AUX = 256


def _encode_body(x_ref, w_ref, o_ref):
    o_ref[...] = jnp.maximum(
        jax.lax.dot_general(x_ref[...], w_ref[...], (((1,), (1,)), ((), ())),
                            preferred_element_type=jnp.float32), 0.0)


def _encode(x, w_enc, bn):
    b, d = x.shape
    h = w_enc.shape[0]
    return pl.pallas_call(
        _encode_body,
        grid=(h // bn,),
        in_specs=[
            pl.BlockSpec((b, d), lambda j: (0, 0)),
            pl.BlockSpec((bn, d), lambda j: (j, 0)),
        ],
        out_specs=pl.BlockSpec((b, bn), lambda j: (0, j)),
        out_shape=jax.ShapeDtypeStruct((b, h), jnp.float32),
        compiler_params=pltpu.CompilerParams(
            dimension_semantics=("parallel",)),
    )(x, w_enc)


def _topk_mask_body(h, enc_ref, em_ref, cnt_ref):
    v = enc_ref[...]
    u = jax.lax.bitcast_convert_type(v, jnp.int32)
    u = jnp.maximum(u, 0)
    bm = v.shape[0]

    def step_t(i, t):
        cand = t | (jnp.int32(1) << (jnp.int32(30) - i))
        cnt = jnp.sum((u >= cand).astype(jnp.int32), axis=1, keepdims=True)
        return jnp.where(cnt >= _TOPK, cand, t)

    t = jax.lax.fori_loop(0, 31, step_t, jnp.zeros((bm, 1), jnp.int32),
                          unroll=True)

    gt = u > t
    n_eq = _TOPK - jnp.sum(gt.astype(jnp.int32), axis=1, keepdims=True)
    eq = u == t
    idx = jax.lax.broadcasted_iota(jnp.int32, (bm, h), 1)

    def step_p(i, p):
        cand = p + (jnp.int32(1) << (jnp.int32(12) - i))
        cnt = jnp.sum((eq & (idx < cand)).astype(jnp.int32), axis=1,
                      keepdims=True)
        return jnp.where(cnt < n_eq, cand, p)

    p = jax.lax.fori_loop(0, 13, step_p, jnp.zeros((bm, 1), jnp.int32),
                          unroll=True)

    mask = gt | (eq & (idx <= p))
    em_ref[...] = v * mask.astype(jnp.float32)
    cnt_ref[...] = jnp.sum(mask.astype(jnp.float32), axis=0,
                           keepdims=True)[None]


def _topk_mask(encoded, bm):
    b, h = encoded.shape
    nb = b // bm
    return pl.pallas_call(
        functools.partial(_topk_mask_body, h),
        grid=(nb,),
        in_specs=[pl.BlockSpec((bm, h), lambda i: (i, 0))],
        out_specs=[
            pl.BlockSpec((bm, h), lambda i: (i, 0)),
            pl.BlockSpec((1, 1, h), lambda i: (i, 0, 0)),
        ],
        out_shape=[
            jax.ShapeDtypeStruct((b, h), jnp.float32),
            jax.ShapeDtypeStruct((nb, 1, h), jnp.float32),
        ],
        compiler_params=pltpu.CompilerParams(
            dimension_semantics=("parallel",)),
    )(encoded)


def _aux_cols_body(b, h, cnt_ref, mask_ref, s_ref, key_ref):
    active = jnp.sum(cnt_ref[...], axis=(0, 1))[None, :]
    inact = (jnp.float32(b) - active).astype(jnp.int32)
    idx = jax.lax.broadcasted_iota(jnp.int32, (1, h), 1)
    key_ref[...] = inact * jnp.int32(h) + (jnp.int32(h - 1) - idx)
    mask_ref[...] = jnp.zeros((1, h), jnp.float32)

    def step(i, carry):
        key = key_ref[...]
        kmax = jnp.max(key)
        hit = key == kmax
        key_ref[...] = jnp.where(hit, jnp.int32(-1), key)
        mask_ref[...] = jnp.where(hit, jnp.float32(1.0), mask_ref[...])
        s_ref[pl.ds(i, 1), :] = hit.astype(jnp.float32)
        return carry

    jax.lax.fori_loop(0, _TOPK_AUX, step, jnp.int32(0), unroll=8)


def _aux_cols(cnt, b):
    nb, _, h = cnt.shape
    return pl.pallas_call(
        functools.partial(_aux_cols_body, b, h),
        grid=(1,),
        in_specs=[pl.BlockSpec((nb, 1, h), lambda i: (0, 0, 0))],
        out_specs=[
            pl.BlockSpec((1, h), lambda i: (0, 0)),
            pl.BlockSpec((_TOPK_AUX, h), lambda i: (0, 0)),
        ],
        out_shape=[
            jax.ShapeDtypeStruct((1, h), jnp.float32),
            jax.ShapeDtypeStruct((_TOPK_AUX, h), jnp.float32),
        ],
        scratch_shapes=[pltpu.VMEM((1, h), jnp.int32)],
    )(cnt)


def _decode_body(nk, em_ref, w_ref, cm_ref, s_ref, b_ref,
                 dec_ref, emaux_ref, ec_ref, wc_ref,
                 accd_ref, acce_ref, accw_ref):
    i = pl.program_id(0)
    k = pl.program_id(1)
    em = em_ref[...]
    w = w_ref[...]
    s = s_ref[...]

    @pl.when(k == 0)
    def _():
        accd_ref[...] = jnp.zeros_like(accd_ref)
        acce_ref[...] = jnp.zeros_like(acce_ref)

    dims = (((1,), (1,)), ((), ()))
    accd_ref[...] += jax.lax.dot_general(
        em, w, dims, preferred_element_type=jnp.float32)
    acce_ref[...] += jax.lax.dot_general(
        em, s, dims, preferred_element_type=jnp.float32)
    emaux_ref[...] = em * cm_ref[...]

    @pl.when(i == 0)
    def _():
        @pl.when(k == 0)
        def _():
            accw_ref[...] = jnp.zeros_like(accw_ref)

        accw_ref[...] += jax.lax.dot_general(
            w, s, dims, preferred_element_type=jnp.float32)

        @pl.when(k == nk - 1)
        def _():
            wc_ref[...] = accw_ref[...]

    @pl.when(k == nk - 1)
    def _():
        dec_ref[...] = accd_ref[...] + b_ref[...]
        ec_ref[...] = acce_ref[...]


def _decode_fused(em, w_dec, colmask, s, bias2, bm, hk):
    b, h = em.shape
    d = w_dec.shape[0]
    nk = h // hk
    grid = (b // bm, nk)
    return pl.pallas_call(
        functools.partial(_decode_body, nk),
        grid=grid,
        in_specs=[
            pl.BlockSpec((bm, hk), lambda i, k: (i, k)),
            pl.BlockSpec((d, hk), lambda i, k: (0, k)),
            pl.BlockSpec((1, hk), lambda i, k: (0, k)),
            pl.BlockSpec((_TOPK_AUX, hk), lambda i, k: (0, k)),
            pl.BlockSpec((1, d), lambda i, k: (0, 0)),
        ],
        out_specs=[
            pl.BlockSpec((bm, d), lambda i, k: (i, 0)),
            pl.BlockSpec((bm, hk), lambda i, k: (i, k)),
            pl.BlockSpec((bm, _TOPK_AUX), lambda i, k: (i, 0)),
            pl.BlockSpec((d, _TOPK_AUX), lambda i, k: (0, 0)),
        ],
        out_shape=[
            jax.ShapeDtypeStruct((b, d), jnp.float32),
            jax.ShapeDtypeStruct((b, h), jnp.float32),
            jax.ShapeDtypeStruct((b, _TOPK_AUX), jnp.float32),
            jax.ShapeDtypeStruct((d, _TOPK_AUX), jnp.float32),
        ],
        scratch_shapes=[
            pltpu.VMEM((bm, d), jnp.float32),
            pltpu.VMEM((bm, _TOPK_AUX), jnp.float32),
            pltpu.VMEM((d, _TOPK_AUX), jnp.float32),
        ],
        compiler_params=pltpu.CompilerParams(
            dimension_semantics=("parallel", "arbitrary")),
    )(em, w_dec, colmask, s, bias2)


def _mm_bias_body(x_ref, w_ref, b_ref, o_ref):
    o_ref[...] = jax.lax.dot_general(
        x_ref[...], w_ref[...], (((1,), (1,)), ((), ())),
        preferred_element_type=jnp.float32) + b_ref[...]


def _matmul_t_bias_1k(x, w, b, bm, bn):
    m, kdim = x.shape
    n = w.shape[0]
    grid = (m // bm, n // bn)
    return pl.pallas_call(
        _mm_bias_body,
        grid=grid,
        in_specs=[
            pl.BlockSpec((bm, kdim), lambda i, j: (i, 0)),
            pl.BlockSpec((bn, kdim), lambda i, j: (j, 0)),
            pl.BlockSpec((1, bn), lambda i, j: (0, j)),
        ],
        out_specs=pl.BlockSpec((bm, bn), lambda i, j: (i, j)),
        out_shape=jax.ShapeDtypeStruct((m, n), jnp.float32),
        compiler_params=pltpu.CompilerParams(
            dimension_semantics=("parallel", "parallel")),
    )(x, w, b)


def kernel(x, bias, W_enc, W_dec):
    b, d = x.shape
    h = W_enc.shape[0]
    bias2 = bias.reshape(1, d)

    encoded = _encode(x, W_enc, min(512, h))
    em, cnt = _topk_mask(encoded, min(256, b))
    colmask, s = _aux_cols(cnt, b)
    decoded, em_aux, ecomp, wcomp = _decode_fused(
        em, W_dec, colmask, s, bias2, min(512, b), min(512, h))
    decoded_aux = _matmul_t_bias_1k(ecomp, wcomp, bias2, min(1024, b),
                                    min(1024, d))
    return (em, decoded, em_aux, decoded_aux)


# speedup vs baseline: 1.1155x; 1.1155x over previous
import functools

import jax
import jax.numpy as jnp
from jax.experimental import pallas as pl
from jax.experimental.pallas import tpu as pltpu

_TOPK = 64
_TOPK_AUX = 256


def _encode_body(x_ref, w_ref, o_ref):
    o_ref[...] = jnp.maximum(
        jax.lax.dot_general(x_ref[...], w_ref[...], (((1,), (1,)), ((), ())),
                            preferred_element_type=jnp.float32), 0.0)


def _encode(x, w_enc, bn):
    b, d = x.shape
    h = w_enc.shape[0]
    return pl.pallas_call(
        _encode_body,
        grid=(h // bn,),
        in_specs=[
            pl.BlockSpec((b, d), lambda j: (0, 0)),
            pl.BlockSpec((bn, d), lambda j: (j, 0)),
        ],
        out_specs=pl.BlockSpec((b, bn), lambda j: (0, j)),
        out_shape=jax.ShapeDtypeStruct((b, h), jnp.float32),
        compiler_params=pltpu.CompilerParams(
            dimension_semantics=("parallel",)),
    )(x, w_enc)


def _topk_mask_body(h, enc_ref, em_ref, cnt_ref):
    v = enc_ref[...]
    u = jax.lax.bitcast_convert_type(v, jnp.int32)
    u = jnp.maximum(u, 0)
    bm = v.shape[0]

    def step_t(i, t):
        cand = t | (jnp.int32(1) << (jnp.int32(30) - i))
        cnt = jnp.sum((u >= cand).astype(jnp.int32), axis=1, keepdims=True)
        return jnp.where(cnt >= _TOPK, cand, t)

    t = jax.lax.fori_loop(0, 31, step_t, jnp.zeros((bm, 1), jnp.int32),
                          unroll=True)

    gt = u > t
    n_eq = _TOPK - jnp.sum(gt.astype(jnp.int32), axis=1, keepdims=True)
    eq = u == t
    idx = jax.lax.broadcasted_iota(jnp.int32, (bm, h), 1)

    def step_p(i, p):
        cand = p + (jnp.int32(1) << (jnp.int32(12) - i))
        cnt = jnp.sum((eq & (idx < cand)).astype(jnp.int32), axis=1,
                      keepdims=True)
        return jnp.where(cnt < n_eq, cand, p)

    p = jax.lax.fori_loop(0, 13, step_p, jnp.zeros((bm, 1), jnp.int32),
                          unroll=True)

    mask = gt | (eq & (idx <= p))
    em_ref[...] = v * mask.astype(jnp.float32)
    cnt_ref[...] = jnp.sum(mask.astype(jnp.float32), axis=0,
                           keepdims=True)[None]


def _topk_mask(encoded, bm):
    b, h = encoded.shape
    nb = b // bm
    return pl.pallas_call(
        functools.partial(_topk_mask_body, h),
        grid=(nb,),
        in_specs=[pl.BlockSpec((bm, h), lambda i: (i, 0))],
        out_specs=[
            pl.BlockSpec((bm, h), lambda i: (i, 0)),
            pl.BlockSpec((1, 1, h), lambda i: (i, 0, 0)),
        ],
        out_shape=[
            jax.ShapeDtypeStruct((b, h), jnp.float32),
            jax.ShapeDtypeStruct((nb, 1, h), jnp.float32),
        ],
        compiler_params=pltpu.CompilerParams(
            dimension_semantics=("parallel",)),
    )(encoded)


def _aux_cols_body(b, h, cnt_ref, mask_ref, s_ref, key_ref):
    active = jnp.sum(cnt_ref[...], axis=(0, 1))[None, :]
    inact = (jnp.float32(b) - active).astype(jnp.int32)
    idx = jax.lax.broadcasted_iota(jnp.int32, (1, h), 1)
    key_ref[...] = inact * jnp.int32(h) + (jnp.int32(h - 1) - idx)
    mask_ref[...] = jnp.zeros((1, h), jnp.float32)

    def step(i, carry):
        key = key_ref[...]
        kmax = jnp.max(key)
        hit = key == kmax
        key_ref[...] = jnp.where(hit, jnp.int32(-1), key)
        mask_ref[...] = jnp.where(hit, jnp.float32(1.0), mask_ref[...])
        s_ref[pl.ds(i, 1), :] = hit.astype(jnp.float32)
        return carry

    jax.lax.fori_loop(0, _TOPK_AUX, step, jnp.int32(0), unroll=8)


def _aux_cols(cnt, b):
    nb, _, h = cnt.shape
    return pl.pallas_call(
        functools.partial(_aux_cols_body, b, h),
        grid=(1,),
        in_specs=[pl.BlockSpec((nb, 1, h), lambda i: (0, 0, 0))],
        out_specs=[
            pl.BlockSpec((1, h), lambda i: (0, 0)),
            pl.BlockSpec((_TOPK_AUX, h), lambda i: (0, 0)),
        ],
        out_shape=[
            jax.ShapeDtypeStruct((1, h), jnp.float32),
            jax.ShapeDtypeStruct((_TOPK_AUX, h), jnp.float32),
        ],
        scratch_shapes=[pltpu.VMEM((1, h), jnp.int32)],
    )(cnt)


def _decode_body(nk, em_ref, w_ref, cm_ref, s_ref, b_ref,
                 dec_ref, emaux_ref, ec_ref, wc_ref,
                 accd_ref, acce_ref, accw_ref):
    i = pl.program_id(0)
    k = pl.program_id(1)
    em = em_ref[...]
    w = w_ref[...]
    s = s_ref[...]

    @pl.when(k == 0)
    def _():
        accd_ref[...] = jnp.zeros_like(accd_ref)
        acce_ref[...] = jnp.zeros_like(acce_ref)

    dims = (((1,), (1,)), ((), ()))
    accd_ref[...] += jax.lax.dot_general(
        em, w, dims, preferred_element_type=jnp.float32)
    acce_ref[...] += jax.lax.dot_general(
        em, s, dims, preferred_element_type=jnp.float32)
    emaux_ref[...] = em * cm_ref[...]

    @pl.when(i == 0)
    def _():
        @pl.when(k == 0)
        def _():
            accw_ref[...] = jnp.zeros_like(accw_ref)

        accw_ref[...] += jax.lax.dot_general(
            w, s, dims, preferred_element_type=jnp.float32)

        @pl.when(k == nk - 1)
        def _():
            wc_ref[...] = accw_ref[...]

    @pl.when(k == nk - 1)
    def _():
        dec_ref[...] = accd_ref[...] + b_ref[...]
        ec_ref[...] = acce_ref[...]


def _decode_fused(em, w_dec, colmask, s, bias2, bm, hk):
    b, h = em.shape
    d = w_dec.shape[0]
    nk = h // hk
    grid = (b // bm, nk)
    return pl.pallas_call(
        functools.partial(_decode_body, nk),
        grid=grid,
        in_specs=[
            pl.BlockSpec((bm, hk), lambda i, k: (i, k)),
            pl.BlockSpec((d, hk), lambda i, k: (0, k)),
            pl.BlockSpec((1, hk), lambda i, k: (0, k)),
            pl.BlockSpec((_TOPK_AUX, hk), lambda i, k: (0, k)),
            pl.BlockSpec((1, d), lambda i, k: (0, 0)),
        ],
        out_specs=[
            pl.BlockSpec((bm, d), lambda i, k: (i, 0)),
            pl.BlockSpec((bm, hk), lambda i, k: (i, k)),
            pl.BlockSpec((bm, _TOPK_AUX), lambda i, k: (i, 0)),
            pl.BlockSpec((d, _TOPK_AUX), lambda i, k: (0, 0)),
        ],
        out_shape=[
            jax.ShapeDtypeStruct((b, d), jnp.float32),
            jax.ShapeDtypeStruct((b, h), jnp.float32),
            jax.ShapeDtypeStruct((b, _TOPK_AUX), jnp.float32),
            jax.ShapeDtypeStruct((d, _TOPK_AUX), jnp.float32),
        ],
        scratch_shapes=[
            pltpu.VMEM((bm, d), jnp.float32),
            pltpu.VMEM((bm, _TOPK_AUX), jnp.float32),
            pltpu.VMEM((d, _TOPK_AUX), jnp.float32),
        ],
        compiler_params=pltpu.CompilerParams(
            dimension_semantics=("parallel", "arbitrary")),
    )(em, w_dec, colmask, s, bias2)


def _mm_bias_body(x_ref, w_ref, b_ref, o_ref):
    o_ref[...] = jax.lax.dot_general(
        x_ref[...], w_ref[...], (((1,), (1,)), ((), ())),
        preferred_element_type=jnp.float32) + b_ref[...]


def _matmul_t_bias_1k(x, w, b, bm, bn):
    m, kdim = x.shape
    n = w.shape[0]
    grid = (m // bm, n // bn)
    return pl.pallas_call(
        _mm_bias_body,
        grid=grid,
        in_specs=[
            pl.BlockSpec((bm, kdim), lambda i, j: (i, 0)),
            pl.BlockSpec((bn, kdim), lambda i, j: (j, 0)),
            pl.BlockSpec((1, bn), lambda i, j: (0, j)),
        ],
        out_specs=pl.BlockSpec((bm, bn), lambda i, j: (i, j)),
        out_shape=jax.ShapeDtypeStruct((m, n), jnp.float32),
        compiler_params=pltpu.CompilerParams(
            dimension_semantics=("parallel", "parallel")),
    )(x, w, b)


def kernel(x, bias, W_enc, W_dec):
    b, d = x.shape
    h = W_enc.shape[0]
    bias2 = bias.reshape(1, d)

    encoded = _encode(x, W_enc, min(512, h))
    em, cnt = _topk_mask(encoded, min(128, b))
    colmask, s = _aux_cols(cnt, b)
    decoded, em_aux, ecomp, wcomp = _decode_fused(
        em, W_dec, colmask, s, bias2, min(512, b), min(512, h))
    decoded_aux = _matmul_t_bias_1k(ecomp, wcomp, bias2, min(1024, b),
                                    min(1024, d))
    return (em, decoded, em_aux, decoded_aux)


# speedup vs baseline: 1.1277x; 1.0110x over previous
import functools

import jax
import jax.numpy as jnp
from jax.experimental import pallas as pl
from jax.experimental.pallas import tpu as pltpu

_TOPK = 64
_TOPK_AUX = 256


def _encode_body(x_ref, w_ref, o_ref):
    o_ref[...] = jnp.maximum(
        jax.lax.dot_general(x_ref[...], w_ref[...], (((1,), (1,)), ((), ())),
                            preferred_element_type=jnp.float32), 0.0)


def _encode(x, w_enc, bn):
    b, d = x.shape
    h = w_enc.shape[0]
    return pl.pallas_call(
        _encode_body,
        grid=(h // bn,),
        in_specs=[
            pl.BlockSpec((b, d), lambda j: (0, 0)),
            pl.BlockSpec((bn, d), lambda j: (j, 0)),
        ],
        out_specs=pl.BlockSpec((b, bn), lambda j: (0, j)),
        out_shape=jax.ShapeDtypeStruct((b, h), jnp.float32),
        compiler_params=pltpu.CompilerParams(
            dimension_semantics=("parallel",)),
    )(x, w_enc)


def _topk_mask_body(h, enc_ref, em_ref, cnt_ref):
    v = enc_ref[...]
    u = jax.lax.bitcast_convert_type(v, jnp.int32)
    u = jnp.maximum(u, 0)
    bm = v.shape[0]

    def step_t(i, t):
        cand = t | (jnp.int32(1) << (jnp.int32(30) - i))
        cnt = jnp.sum((u >= cand).astype(jnp.int32), axis=1, keepdims=True)
        return jnp.where(cnt >= _TOPK, cand, t)

    t = jax.lax.fori_loop(0, 31, step_t, jnp.zeros((bm, 1), jnp.int32),
                          unroll=True)

    gt = u > t
    n_eq = _TOPK - jnp.sum(gt.astype(jnp.int32), axis=1, keepdims=True)
    eq = u == t
    idx = jax.lax.broadcasted_iota(jnp.int32, (bm, h), 1)

    def step_p(i, p):
        cand = p + (jnp.int32(1) << (jnp.int32(12) - i))
        cnt = jnp.sum((eq & (idx < cand)).astype(jnp.int32), axis=1,
                      keepdims=True)
        return jnp.where(cnt < n_eq, cand, p)

    p = jax.lax.fori_loop(0, 13, step_p, jnp.zeros((bm, 1), jnp.int32),
                          unroll=True)

    mask = gt | (eq & (idx <= p))
    em_ref[...] = v * mask.astype(jnp.float32)
    cnt_ref[...] = jnp.sum(mask.astype(jnp.float32), axis=0,
                           keepdims=True)[None]


def _topk_mask(encoded, bm):
    b, h = encoded.shape
    nb = b // bm
    return pl.pallas_call(
        functools.partial(_topk_mask_body, h),
        grid=(nb,),
        in_specs=[pl.BlockSpec((bm, h), lambda i: (i, 0))],
        out_specs=[
            pl.BlockSpec((bm, h), lambda i: (i, 0)),
            pl.BlockSpec((1, 1, h), lambda i: (i, 0, 0)),
        ],
        out_shape=[
            jax.ShapeDtypeStruct((b, h), jnp.float32),
            jax.ShapeDtypeStruct((nb, 1, h), jnp.float32),
        ],
        compiler_params=pltpu.CompilerParams(
            dimension_semantics=("parallel",)),
    )(encoded)


def _aux_cols_body(b, h, cnt_ref, mask_ref, s_ref, key_ref):
    active = jnp.sum(cnt_ref[...], axis=(0, 1))[None, :]
    inact = (jnp.float32(b) - active).astype(jnp.int32)
    idx = jax.lax.broadcasted_iota(jnp.int32, (1, h), 1)
    key_ref[...] = inact * jnp.int32(h) + (jnp.int32(h - 1) - idx)
    mask_ref[...] = jnp.zeros((1, h), jnp.float32)

    def step(i, carry):
        key = key_ref[...]
        kmax = jnp.max(key)
        hit = key == kmax
        key_ref[...] = jnp.where(hit, jnp.int32(-1), key)
        mask_ref[...] = jnp.where(hit, jnp.float32(1.0), mask_ref[...])
        s_ref[pl.ds(i, 1), :] = hit.astype(jnp.float32)
        return carry

    jax.lax.fori_loop(0, _TOPK_AUX, step, jnp.int32(0), unroll=8)


def _aux_cols(cnt, b):
    nb, _, h = cnt.shape
    return pl.pallas_call(
        functools.partial(_aux_cols_body, b, h),
        grid=(1,),
        in_specs=[pl.BlockSpec((nb, 1, h), lambda i: (0, 0, 0))],
        out_specs=[
            pl.BlockSpec((1, h), lambda i: (0, 0)),
            pl.BlockSpec((_TOPK_AUX, h), lambda i: (0, 0)),
        ],
        out_shape=[
            jax.ShapeDtypeStruct((1, h), jnp.float32),
            jax.ShapeDtypeStruct((_TOPK_AUX, h), jnp.float32),
        ],
        scratch_shapes=[pltpu.VMEM((1, h), jnp.int32)],
    )(cnt)


def _decode_body(nk, em_ref, w_ref, cm_ref, s_ref, b_ref,
                 dec_ref, emaux_ref, ec_ref, wc_ref,
                 accd_ref, acce_ref, accw_ref):
    i = pl.program_id(0)
    k = pl.program_id(1)
    em = em_ref[...]
    w = w_ref[...]
    s = s_ref[...]

    @pl.when(k == 0)
    def _():
        accd_ref[...] = jnp.zeros_like(accd_ref)
        acce_ref[...] = jnp.zeros_like(acce_ref)

    dims = (((1,), (1,)), ((), ()))
    accd_ref[...] += jax.lax.dot_general(
        em, w, dims, preferred_element_type=jnp.float32)
    acce_ref[...] += jax.lax.dot_general(
        em, s, dims, preferred_element_type=jnp.float32)
    emaux_ref[...] = em * cm_ref[...]

    @pl.when(i == 0)
    def _():
        @pl.when(k == 0)
        def _():
            accw_ref[...] = jnp.zeros_like(accw_ref)

        accw_ref[...] += jax.lax.dot_general(
            w, s, dims, preferred_element_type=jnp.float32)

        @pl.when(k == nk - 1)
        def _():
            wc_ref[...] = accw_ref[...]

    @pl.when(k == nk - 1)
    def _():
        dec_ref[...] = accd_ref[...] + b_ref[...]
        ec_ref[...] = acce_ref[...]


def _decode_fused(em, w_dec, colmask, s, bias2, bm, hk):
    b, h = em.shape
    d = w_dec.shape[0]
    nk = h // hk
    grid = (b // bm, nk)
    return pl.pallas_call(
        functools.partial(_decode_body, nk),
        grid=grid,
        in_specs=[
            pl.BlockSpec((bm, hk), lambda i, k: (i, k)),
            pl.BlockSpec((d, hk), lambda i, k: (0, k)),
            pl.BlockSpec((1, hk), lambda i, k: (0, k)),
            pl.BlockSpec((_TOPK_AUX, hk), lambda i, k: (0, k)),
            pl.BlockSpec((1, d), lambda i, k: (0, 0)),
        ],
        out_specs=[
            pl.BlockSpec((bm, d), lambda i, k: (i, 0)),
            pl.BlockSpec((bm, hk), lambda i, k: (i, k)),
            pl.BlockSpec((bm, _TOPK_AUX), lambda i, k: (i, 0)),
            pl.BlockSpec((d, _TOPK_AUX), lambda i, k: (0, 0)),
        ],
        out_shape=[
            jax.ShapeDtypeStruct((b, d), jnp.float32),
            jax.ShapeDtypeStruct((b, h), jnp.float32),
            jax.ShapeDtypeStruct((b, _TOPK_AUX), jnp.float32),
            jax.ShapeDtypeStruct((d, _TOPK_AUX), jnp.float32),
        ],
        scratch_shapes=[
            pltpu.VMEM((bm, d), jnp.float32),
            pltpu.VMEM((bm, _TOPK_AUX), jnp.float32),
            pltpu.VMEM((d, _TOPK_AUX), jnp.float32),
        ],
        compiler_params=pltpu.CompilerParams(
            dimension_semantics=("parallel", "arbitrary")),
    )(em, w_dec, colmask, s, bias2)


def _mm_bias_body(x_ref, w_ref, b_ref, o_ref):
    o_ref[...] = jax.lax.dot_general(
        x_ref[...], w_ref[...], (((1,), (1,)), ((), ())),
        preferred_element_type=jnp.float32) + b_ref[...]


def _matmul_t_bias_1k(x, w, b, bm, bn):
    m, kdim = x.shape
    n = w.shape[0]
    grid = (m // bm, n // bn)
    return pl.pallas_call(
        _mm_bias_body,
        grid=grid,
        in_specs=[
            pl.BlockSpec((bm, kdim), lambda i, j: (i, 0)),
            pl.BlockSpec((bn, kdim), lambda i, j: (j, 0)),
            pl.BlockSpec((1, bn), lambda i, j: (0, j)),
        ],
        out_specs=pl.BlockSpec((bm, bn), lambda i, j: (i, j)),
        out_shape=jax.ShapeDtypeStruct((m, n), jnp.float32),
        compiler_params=pltpu.CompilerParams(
            dimension_semantics=("parallel", "parallel")),
    )(x, w, b)


def kernel(x, bias, W_enc, W_dec):
    b, d = x.shape
    h = W_enc.shape[0]
    bias2 = bias.reshape(1, d)

    encoded = _encode(x, W_enc, min(512, h))
    em, cnt = _topk_mask(encoded, min(128, b))
    colmask, s = _aux_cols(cnt, b)
    decoded, em_aux, ecomp, wcomp = _decode_fused(
        em, W_dec, colmask, s, bias2, min(512, b), min(1024, h))
    decoded_aux = _matmul_t_bias_1k(ecomp, wcomp, bias2, min(1024, b),
                                    min(1024, d))
    return (em, decoded, em_aux, decoded_aux)


# speedup vs baseline: 1.1316x; 1.0034x over previous
import functools

import jax
import jax.numpy as jnp
from jax.experimental import pallas as pl
from jax.experimental.pallas import tpu as pltpu

_TOPK = 64
_TOPK_AUX = 256


def _encode_body(x_ref, w_ref, o_ref):
    o_ref[...] = jnp.maximum(
        jax.lax.dot_general(x_ref[...], w_ref[...], (((1,), (1,)), ((), ())),
                            preferred_element_type=jnp.float32), 0.0)


def _encode(x, w_enc, bn):
    b, d = x.shape
    h = w_enc.shape[0]
    return pl.pallas_call(
        _encode_body,
        grid=(h // bn,),
        in_specs=[
            pl.BlockSpec((b, d), lambda j: (0, 0)),
            pl.BlockSpec((bn, d), lambda j: (j, 0)),
        ],
        out_specs=pl.BlockSpec((b, bn), lambda j: (0, j)),
        out_shape=jax.ShapeDtypeStruct((b, h), jnp.float32),
        compiler_params=pltpu.CompilerParams(
            dimension_semantics=("parallel",)),
    )(x, w_enc)


def _topk_mask_body(h, enc_ref, em_ref, cnt_ref):
    v = enc_ref[...]
    u = jax.lax.bitcast_convert_type(v, jnp.int32)
    u = jnp.maximum(u, 0)
    bm = v.shape[0]

    def step_t(i, t):
        cand = t | (jnp.int32(1) << (jnp.int32(30) - i))
        cnt = jnp.sum((u >= cand).astype(jnp.int32), axis=1, keepdims=True)
        return jnp.where(cnt >= _TOPK, cand, t)

    t = jax.lax.fori_loop(0, 31, step_t, jnp.zeros((bm, 1), jnp.int32),
                          unroll=True)

    gt = u > t
    n_eq = _TOPK - jnp.sum(gt.astype(jnp.int32), axis=1, keepdims=True)
    eq = u == t
    idx = jax.lax.broadcasted_iota(jnp.int32, (bm, h), 1)

    def step_p(i, p):
        cand = p + (jnp.int32(1) << (jnp.int32(12) - i))
        cnt = jnp.sum((eq & (idx < cand)).astype(jnp.int32), axis=1,
                      keepdims=True)
        return jnp.where(cnt < n_eq, cand, p)

    p = jax.lax.fori_loop(0, 13, step_p, jnp.zeros((bm, 1), jnp.int32),
                          unroll=True)

    mask = gt | (eq & (idx <= p))
    em_ref[...] = v * mask.astype(jnp.float32)
    cnt_ref[...] = jnp.sum(mask.astype(jnp.float32), axis=0,
                           keepdims=True)[None]


def _topk_mask(encoded, bm):
    b, h = encoded.shape
    nb = b // bm
    return pl.pallas_call(
        functools.partial(_topk_mask_body, h),
        grid=(nb,),
        in_specs=[pl.BlockSpec((bm, h), lambda i: (i, 0))],
        out_specs=[
            pl.BlockSpec((bm, h), lambda i: (i, 0)),
            pl.BlockSpec((1, 1, h), lambda i: (i, 0, 0)),
        ],
        out_shape=[
            jax.ShapeDtypeStruct((b, h), jnp.float32),
            jax.ShapeDtypeStruct((nb, 1, h), jnp.float32),
        ],
        compiler_params=pltpu.CompilerParams(
            dimension_semantics=("parallel",)),
    )(encoded)


def _aux_cols_body(b, h, cnt_ref, mask_ref, s_ref, key_ref):
    active = jnp.sum(cnt_ref[...], axis=(0, 1))[None, :]
    inact = (jnp.float32(b) - active).astype(jnp.int32)
    idx = jax.lax.broadcasted_iota(jnp.int32, (1, h), 1)
    key_ref[...] = inact * jnp.int32(h) + (jnp.int32(h - 1) - idx)
    mask_ref[...] = jnp.zeros((1, h), jnp.float32)

    def step(i, carry):
        key = key_ref[...]
        kmax = jnp.max(key)
        hit = key == kmax
        key_ref[...] = jnp.where(hit, jnp.int32(-1), key)
        mask_ref[...] = jnp.where(hit, jnp.float32(1.0), mask_ref[...])
        s_ref[pl.ds(i, 1), :] = hit.astype(jnp.float32)
        return carry

    jax.lax.fori_loop(0, _TOPK_AUX, step, jnp.int32(0), unroll=8)


def _aux_cols(cnt, b):
    nb, _, h = cnt.shape
    return pl.pallas_call(
        functools.partial(_aux_cols_body, b, h),
        grid=(1,),
        in_specs=[pl.BlockSpec((nb, 1, h), lambda i: (0, 0, 0))],
        out_specs=[
            pl.BlockSpec((1, h), lambda i: (0, 0)),
            pl.BlockSpec((_TOPK_AUX, h), lambda i: (0, 0)),
        ],
        out_shape=[
            jax.ShapeDtypeStruct((1, h), jnp.float32),
            jax.ShapeDtypeStruct((_TOPK_AUX, h), jnp.float32),
        ],
        scratch_shapes=[pltpu.VMEM((1, h), jnp.int32)],
    )(cnt)


def _decode_body(nk, em_ref, w_ref, cm_ref, s_ref, b_ref,
                 dec_ref, emaux_ref, ec_ref, wc_ref,
                 accd_ref, acce_ref, accw_ref):
    i = pl.program_id(0)
    k = pl.program_id(1)
    em = em_ref[...]
    w = w_ref[...]
    s = s_ref[...]

    @pl.when(k == 0)
    def _():
        accd_ref[...] = jnp.zeros_like(accd_ref)
        acce_ref[...] = jnp.zeros_like(acce_ref)

    dims = (((1,), (1,)), ((), ()))
    accd_ref[...] += jax.lax.dot_general(
        em, w, dims, preferred_element_type=jnp.float32)
    acce_ref[...] += jax.lax.dot_general(
        em, s, dims, preferred_element_type=jnp.float32)
    emaux_ref[...] = em * cm_ref[...]

    @pl.when(i == 0)
    def _():
        @pl.when(k == 0)
        def _():
            accw_ref[...] = jnp.zeros_like(accw_ref)

        accw_ref[...] += jax.lax.dot_general(
            w, s, dims, preferred_element_type=jnp.float32)

        @pl.when(k == nk - 1)
        def _():
            wc_ref[...] = accw_ref[...]

    @pl.when(k == nk - 1)
    def _():
        dec_ref[...] = accd_ref[...] + b_ref[...]
        ec_ref[...] = acce_ref[...]


def _decode_fused(em, w_dec, colmask, s, bias2, bm, hk):
    b, h = em.shape
    d = w_dec.shape[0]
    nk = h // hk
    grid = (b // bm, nk)
    return pl.pallas_call(
        functools.partial(_decode_body, nk),
        grid=grid,
        in_specs=[
            pl.BlockSpec((bm, hk), lambda i, k: (i, k)),
            pl.BlockSpec((d, hk), lambda i, k: (0, k)),
            pl.BlockSpec((1, hk), lambda i, k: (0, k)),
            pl.BlockSpec((_TOPK_AUX, hk), lambda i, k: (0, k)),
            pl.BlockSpec((1, d), lambda i, k: (0, 0)),
        ],
        out_specs=[
            pl.BlockSpec((bm, d), lambda i, k: (i, 0)),
            pl.BlockSpec((bm, hk), lambda i, k: (i, k)),
            pl.BlockSpec((bm, _TOPK_AUX), lambda i, k: (i, 0)),
            pl.BlockSpec((d, _TOPK_AUX), lambda i, k: (0, 0)),
        ],
        out_shape=[
            jax.ShapeDtypeStruct((b, d), jnp.float32),
            jax.ShapeDtypeStruct((b, h), jnp.float32),
            jax.ShapeDtypeStruct((b, _TOPK_AUX), jnp.float32),
            jax.ShapeDtypeStruct((d, _TOPK_AUX), jnp.float32),
        ],
        scratch_shapes=[
            pltpu.VMEM((bm, d), jnp.float32),
            pltpu.VMEM((bm, _TOPK_AUX), jnp.float32),
            pltpu.VMEM((d, _TOPK_AUX), jnp.float32),
        ],
        compiler_params=pltpu.CompilerParams(
            dimension_semantics=("parallel", "arbitrary")),
    )(em, w_dec, colmask, s, bias2)


def _mm_bias_body(x_ref, w_ref, b_ref, o_ref):
    o_ref[...] = jax.lax.dot_general(
        x_ref[...], w_ref[...], (((1,), (1,)), ((), ())),
        preferred_element_type=jnp.float32) + b_ref[...]


def _matmul_t_bias_1k(x, w, b, bm, bn):
    m, kdim = x.shape
    n = w.shape[0]
    grid = (m // bm, n // bn)
    return pl.pallas_call(
        _mm_bias_body,
        grid=grid,
        in_specs=[
            pl.BlockSpec((bm, kdim), lambda i, j: (i, 0)),
            pl.BlockSpec((bn, kdim), lambda i, j: (j, 0)),
            pl.BlockSpec((1, bn), lambda i, j: (0, j)),
        ],
        out_specs=pl.BlockSpec((bm, bn), lambda i, j: (i, j)),
        out_shape=jax.ShapeDtypeStruct((m, n), jnp.float32),
        compiler_params=pltpu.CompilerParams(
            dimension_semantics=("parallel", "parallel")),
    )(x, w, b)


def kernel(x, bias, W_enc, W_dec):
    b, d = x.shape
    h = W_enc.shape[0]
    bias2 = bias.reshape(1, d)

    encoded = _encode(x, W_enc, min(1024, h))
    em, cnt = _topk_mask(encoded, min(128, b))
    colmask, s = _aux_cols(cnt, b)
    decoded, em_aux, ecomp, wcomp = _decode_fused(
        em, W_dec, colmask, s, bias2, min(512, b), min(1024, h))
    decoded_aux = _matmul_t_bias_1k(ecomp, wcomp, bias2, min(1024, b),
                                    min(1024, d))
    return (em, decoded, em_aux, decoded_aux)
